# baseline (speedup 1.0000x reference)
import jax
import jax.numpy as jnp
from jax import lax
from jax.experimental import pallas as pl
from jax.experimental.pallas import tpu as pltpu
from jax.experimental.pallas import tpu_sc as plsc

D = 768
H = 12
HD = 64
HID = 576
NE = 23
TOPK = 3
NEP = 128
T = 2048
RB = 256
QB = 512
GB = 64
MAXBLKS = 119
MAXROWS = MAXBLKS * GB
NA = T * TOPK
NEG = -1e30

NC, NS = 2, 16
NW = NC * NS
PROWS = NA // NW
PCH = 96
PNCH = PROWS // PCH


def _ln(x, g, b, eps=1e-5):
    mu = jnp.mean(x, -1, keepdims=True)
    xc = x - mu
    var = jnp.mean(xc * xc, -1, keepdims=True)
    return xc * lax.rsqrt(var + eps) * g + b


def _gelu(x):
    return 0.5 * x * (1.0 + lax.erf(x * (2.0 ** -0.5)))


def _ln_qkv_kernel(x_ref, g_ref, b_ref, w_ref, o_ref):
    xa = _ln(x_ref[...], g_ref[...], b_ref[...]).astype(jnp.bfloat16)
    o_ref[...] = jnp.dot(xa, w_ref[...],
                         preferred_element_type=jnp.float32).astype(jnp.bfloat16)


def _attn_kernel(q_ref, kv_ref, o_ref):
    for h in range(H):
        q = q_ref[:, h * HD:(h + 1) * HD]
        k = kv_ref[:, D + h * HD:D + (h + 1) * HD]
        v = kv_ref[:, 2 * D + h * HD:2 * D + (h + 1) * HD]
        s = lax.dot_general(q, k, (((1,), (1,)), ((), ())),
                            preferred_element_type=jnp.float32) * (HD ** -0.5)
        m = jnp.max(s, -1, keepdims=True)
        e = jnp.exp(s - m)
        p = (e / jnp.sum(e, -1, keepdims=True)).astype(jnp.bfloat16)
        o_ref[:, h * HD:(h + 1) * HD] = jnp.dot(
            p, v, preferred_element_type=jnp.float32).astype(jnp.bfloat16)


def _proj_gate_kernel(ao_ref, x_ref, pw_ref, pb_ref, g2_ref, b2_ref,
                      gw_ref, gb_ref, x1_ref, xm_ref, gate_ref):
    x1 = (x_ref[...]
          + jnp.dot(ao_ref[...], pw_ref[...], preferred_element_type=jnp.float32)
          + pb_ref[...])
    x1_ref[...] = x1
    xm = _ln(x1, g2_ref[...], b2_ref[...])
    xm_ref[...] = xm
    logits = jnp.dot(xm, gw_ref[...], preferred_element_type=jnp.float32) + gb_ref[...]
    gate_ref[...] = jax.nn.sigmoid(logits)


def _router_kernel(gate_ref, bias_ref, idx_ref, w_ref, rank_ref, cnt_ref,
                   p_ref, run_ref):
    g = gate_ref[...]
    lane = lax.broadcasted_iota(jnp.int32, g.shape, 1)
    s = g + bias_ref[...]
    idxs, ws = [], []
    for _ in range(TOPK):
        m = jnp.max(s, -1, keepdims=True)
        hit = s == m
        idx = jnp.min(jnp.where(hit, lane, NEP), -1, keepdims=True)
        w = jnp.sum(jnp.where(lane == idx, g, 0.0), -1, keepdims=True)
        s = jnp.where(lane == idx, NEG, s)
        idxs.append(idx)
        ws.append(w)
    wsum = ws[0] + ws[1] + ws[2]
    idx_ref[...] = jnp.where(lane == 0, idxs[0],
                   jnp.where(lane == 1, idxs[1],
                   jnp.where(lane == 2, idxs[2], 0)))
    w_ref[...] = jnp.where(lane == 0, ws[0],
                 jnp.where(lane == 1, ws[1],
                 jnp.where(lane == 2, ws[2], 0.0))) / wsum

    @pl.when(pl.program_id(0) == 0)
    def _():
        run_ref[...] = jnp.zeros_like(run_ref)

    oh3 = ((lane == idxs[0]) | (lane == idxs[1]) | (lane == idxs[2]))
    oh3 = oh3.astype(jnp.bfloat16)
    r = lax.broadcasted_iota(jnp.int32, (RB, RB), 0)
    c = lax.broadcasted_iota(jnp.int32, (RB, RB), 1)
    ltri = (c < r).astype(jnp.bfloat16)
    cum = jnp.dot(ltri, oh3, preferred_element_type=jnp.float32)
    tot = cum + run_ref[...]
    ranks = [jnp.sum(jnp.where(lane == idxs[k], tot, 0.0), -1, keepdims=True)
             for k in range(TOPK)]
    rank_ref[...] = jnp.where(lane == 0, ranks[0],
                    jnp.where(lane == 1, ranks[1],
                    jnp.where(lane == 2, ranks[2], 0.0)))

    blockcnt = jnp.sum(oh3.astype(jnp.float32), 0, keepdims=True)
    run_ref[...] = run_ref[...] + blockcnt
    cnt_ref[0] = blockcnt
    gn = g / jnp.sum(g, -1, keepdims=True)
    p_ref[0] = jnp.sum(gn, 0, keepdims=True)


def _dest_kernel(idx_ref, rank_ref, po_ref, dga_ref):
    idxv = idx_ref[...]
    rankv = rank_ref[...]
    po = po_ref[...]
    lane = lax.broadcasted_iota(jnp.int32, idxv.shape, 1)
    dests = []
    for k in range(TOPK):
        ek = jnp.sum(jnp.where(lane == k, idxv, 0), -1, keepdims=True)
        rk = jnp.sum(jnp.where(lane == k, rankv, 0.0), -1, keepdims=True)
        pk = jnp.sum(jnp.where(lane == ek, po, 0.0), -1, keepdims=True)
        dests.append((pk + rk).astype(jnp.int32))
    dga_ref[...] = jnp.concatenate(dests, axis=1)


def _gmm_kernel(be_ref, xd_ref, w1_ref, b1_ref, w2_ref, b2_ref, o_ref):
    xb = xd_ref[...].astype(jnp.bfloat16)
    h = jnp.dot(xb, w1_ref[0], preferred_element_type=jnp.float32) + b1_ref[0]
    h = _gelu(h).astype(jnp.bfloat16)
    o_ref[...] = jnp.dot(h, w2_ref[0], preferred_element_type=jnp.float32) + b2_ref[0]


def _shared_kernel(xm_ref, x1_ref, w1_ref, b1_ref, w2_ref, b2_ref, o_ref):
    xb = xm_ref[...].astype(jnp.bfloat16)
    h = jnp.dot(xb, w1_ref[...], preferred_element_type=jnp.float32) + b1_ref[...]
    h = _gelu(h).astype(jnp.bfloat16)
    o_ref[...] = (x1_ref[...]
                  + jnp.dot(h, w2_ref[...], preferred_element_type=jnp.float32)
                  + b2_ref[...])


def _dispatch_body(xm_hbm, tok_hbm, dga_hbm, xd_hbm, idx_v, dst_v, rows_v, sem):
    c = lax.axis_index("c")
    s = lax.axis_index("s")
    base = (c * NS + s) * PROWS
    for j in range(PNCH):
        off = base + j * PCH
        pltpu.sync_copy(tok_hbm.at[pl.ds(off, PCH)], idx_v)
        pltpu.async_copy(xm_hbm.at[idx_v], rows_v, sem).wait()
        pltpu.sync_copy(dga_hbm.at[pl.ds(off, PCH)], dst_v)
        pltpu.async_copy(rows_v, xd_hbm.at[dst_v], sem).wait()


def _permute_body(eout_hbm, dga_hbm, ep_hbm, idx_v, rows_v, sem):
    c = lax.axis_index("c")
    s = lax.axis_index("s")
    base = (c * NS + s) * PROWS
    for j in range(PNCH):
        off = base + j * PCH
        pltpu.sync_copy(dga_hbm.at[pl.ds(off, PCH)], idx_v)
        pltpu.async_copy(eout_hbm.at[idx_v], rows_v, sem).wait()
        pltpu.sync_copy(rows_v, ep_hbm.at[pl.ds(off, PCH)])


def _comb_kernel(ep_ref, part_ref, w_ref, o_ref):
    e3 = ep_ref[...].reshape(RB, TOPK, D)
    w = w_ref[...]
    o_ref[...] = (part_ref[...]
                  + w[:, 0:1] * e3[:, 0]
                  + w[:, 1:2] * e3[:, 1]
                  + w[:, 2:3] * e3[:, 2])


def kernel(x, norm1_g, norm1_b, qkv_w, proj_w, proj_b, norm2_g, norm2_b,
           gate_w, gate_b, e1_w, e1_b, e2_w, e2_b, s1_w, s1_b, s2_w, s2_b,
           moe_bias):
    B, N, C = x.shape
    x2 = x.reshape(T, D)
    f32 = jnp.float32
    bf16 = jnp.bfloat16

    qkv = pl.pallas_call(
        _ln_qkv_kernel,
        grid=(T // RB,),
        in_specs=[
            pl.BlockSpec((RB, D), lambda i: (i, 0)),
            pl.BlockSpec((1, D), lambda i: (0, 0)),
            pl.BlockSpec((1, D), lambda i: (0, 0)),
            pl.BlockSpec((D, 3 * D), lambda i: (0, 0)),
        ],
        out_specs=pl.BlockSpec((RB, 3 * D), lambda i: (i, 0)),
        out_shape=jax.ShapeDtypeStruct((T, 3 * D), bf16),
    )(x2, norm1_g.reshape(1, D), norm1_b.reshape(1, D), qkv_w.T.astype(bf16))

    ao = pl.pallas_call(
        _attn_kernel,
        grid=(T // QB,),
        in_specs=[
            pl.BlockSpec((QB, 3 * D), lambda i: (i, 0)),
            pl.BlockSpec((T, 3 * D), lambda i: (0, 0)),
        ],
        out_specs=pl.BlockSpec((QB, D), lambda i: (i, 0)),
        out_shape=jax.ShapeDtypeStruct((T, D), bf16),
    )(qkv, qkv)

    gw = jnp.zeros((D, NEP), f32).at[:, :NE].set(gate_w.T)
    gb = jnp.full((1, NEP), NEG, f32).at[0, :NE].set(gate_b)
    x1, xm, gate = pl.pallas_call(
        _proj_gate_kernel,
        grid=(T // RB,),
        in_specs=[
            pl.BlockSpec((RB, D), lambda i: (i, 0)),
            pl.BlockSpec((RB, D), lambda i: (i, 0)),
            pl.BlockSpec((D, D), lambda i: (0, 0)),
            pl.BlockSpec((1, D), lambda i: (0, 0)),
            pl.BlockSpec((1, D), lambda i: (0, 0)),
            pl.BlockSpec((1, D), lambda i: (0, 0)),
            pl.BlockSpec((D, NEP), lambda i: (0, 0)),
            pl.BlockSpec((1, NEP), lambda i: (0, 0)),
        ],
        out_specs=[
            pl.BlockSpec((RB, D), lambda i: (i, 0)),
            pl.BlockSpec((RB, D), lambda i: (i, 0)),
            pl.BlockSpec((RB, NEP), lambda i: (i, 0)),
        ],
        out_shape=[
            jax.ShapeDtypeStruct((T, D), f32),
            jax.ShapeDtypeStruct((T, D), f32),
            jax.ShapeDtypeStruct((T, NEP), f32),
        ],
    )(ao, x2, proj_w.T.astype(bf16), proj_b.reshape(1, D), norm2_g.reshape(1, D),
      norm2_b.reshape(1, D), gw, gb)

    mb = jnp.full((1, NEP), NEG, f32).at[0, :NE].set(moe_bias)
    idx_o, w_o, rank_o, cnt_p, p_p = pl.pallas_call(
        _router_kernel,
        grid=(T // RB,),
        in_specs=[
            pl.BlockSpec((RB, NEP), lambda i: (i, 0)),
            pl.BlockSpec((1, NEP), lambda i: (0, 0)),
        ],
        out_specs=[
            pl.BlockSpec((RB, NEP), lambda i: (i, 0)),
            pl.BlockSpec((RB, NEP), lambda i: (i, 0)),
            pl.BlockSpec((RB, NEP), lambda i: (i, 0)),
            pl.BlockSpec((1, 1, NEP), lambda i: (i, 0, 0)),
            pl.BlockSpec((1, 1, NEP), lambda i: (i, 0, 0)),
        ],
        out_shape=[
            jax.ShapeDtypeStruct((T, NEP), jnp.int32),
            jax.ShapeDtypeStruct((T, NEP), f32),
            jax.ShapeDtypeStruct((T, NEP), f32),
            jax.ShapeDtypeStruct((T // RB, 1, NEP), f32),
            jax.ShapeDtypeStruct((T // RB, 1, NEP), f32),
        ],
        scratch_shapes=[pltpu.VMEM((1, NEP), f32)],
    )(gate, mb)

    counts = cnt_p[:, 0, :NE].sum(0).astype(jnp.int32)

    padc = ((counts + GB - 1) // GB) * GB
    incl = jnp.cumsum(padc)
    pad_off = jnp.concatenate([jnp.zeros((1,), jnp.int32), incl[:-1]])
    po128 = jnp.zeros((1, NEP), f32).at[0, :NE].set(pad_off.astype(f32))
    blk_start = jnp.arange(MAXBLKS, dtype=jnp.int32) * GB
    block_expert = jnp.minimum(
        jnp.searchsorted(incl, blk_start, side='right').astype(jnp.int32), NE - 1)

    dga = pl.pallas_call(
        _dest_kernel,
        grid=(T // RB,),
        in_specs=[
            pl.BlockSpec((RB, NEP), lambda i: (i, 0)),
            pl.BlockSpec((RB, NEP), lambda i: (i, 0)),
            pl.BlockSpec((1, NEP), lambda i: (0, 0)),
        ],
        out_specs=pl.BlockSpec((RB, TOPK), lambda i: (i, 0)),
        out_shape=jax.ShapeDtypeStruct((T, TOPK), jnp.int32),
    )(idx_o, rank_o, po128)

    dga_flat = dga.reshape(NA)
    tok3 = jnp.repeat(jnp.arange(T, dtype=jnp.int32), TOPK)

    mesh = plsc.VectorSubcoreMesh(core_axis_name="c", subcore_axis_name="s")

    xd = pl.kernel(
        _dispatch_body,
        out_type=jax.ShapeDtypeStruct((MAXROWS, D), f32),
        mesh=mesh,
        scratch_types=[
            pltpu.VMEM((PCH,), jnp.int32),
            pltpu.VMEM((PCH,), jnp.int32),
            pltpu.VMEM((PCH, D), f32),
            pltpu.SemaphoreType.DMA,
        ],
    )(xm, tok3, dga_flat)

    eout = pl.pallas_call(
        _gmm_kernel,
        grid_spec=pltpu.PrefetchScalarGridSpec(
            num_scalar_prefetch=1,
            grid=(MAXBLKS,),
            in_specs=[
                pl.BlockSpec((GB, D), lambda i, be: (i, 0)),
                pl.BlockSpec((1, D, HID), lambda i, be: (be[i], 0, 0)),
                pl.BlockSpec((1, 1, HID), lambda i, be: (be[i], 0, 0)),
                pl.BlockSpec((1, HID, D), lambda i, be: (be[i], 0, 0)),
                pl.BlockSpec((1, 1, D), lambda i, be: (be[i], 0, 0)),
            ],
            out_specs=pl.BlockSpec((GB, D), lambda i, be: (i, 0)),
        ),
        out_shape=jax.ShapeDtypeStruct((MAXROWS, D), f32),
    )(block_expert, xd, e1_w.transpose(0, 2, 1).astype(bf16),
      e1_b.reshape(NE, 1, HID), e2_w.transpose(0, 2, 1).astype(bf16),
      e2_b.reshape(NE, 1, D))

    part = pl.pallas_call(
        _shared_kernel,
        grid=(T // RB,),
        in_specs=[
            pl.BlockSpec((RB, D), lambda i: (i, 0)),
            pl.BlockSpec((RB, D), lambda i: (i, 0)),
            pl.BlockSpec((D, HID), lambda i: (0, 0)),
            pl.BlockSpec((1, HID), lambda i: (0, 0)),
            pl.BlockSpec((HID, D), lambda i: (0, 0)),
            pl.BlockSpec((1, D), lambda i: (0, 0)),
        ],
        out_specs=pl.BlockSpec((RB, D), lambda i: (i, 0)),
        out_shape=jax.ShapeDtypeStruct((T, D), f32),
    )(xm, x1, s1_w.T.astype(bf16), s1_b.reshape(1, HID),
      s2_w.T.astype(bf16), s2_b.reshape(1, D))

    eperm = pl.kernel(
        _permute_body,
        out_type=jax.ShapeDtypeStruct((NA, D), f32),
        mesh=mesh,
        scratch_types=[
            pltpu.VMEM((PCH,), jnp.int32),
            pltpu.VMEM((PCH, D), f32),
            pltpu.SemaphoreType.DMA,
        ],
    )(eout, dga_flat)

    out = pl.pallas_call(
        _comb_kernel,
        grid=(T // RB,),
        in_specs=[
            pl.BlockSpec((RB * TOPK, D), lambda i: (i, 0)),
            pl.BlockSpec((RB, D), lambda i: (i, 0)),
            pl.BlockSpec((RB, NEP), lambda i: (i, 0)),
        ],
        out_specs=pl.BlockSpec((RB, D), lambda i: (i, 0)),
        out_shape=jax.ShapeDtypeStruct((T, D), f32),
    )(eperm, part, w_o)

    P = p_p[:, 0, :NE].sum(0) / T
    Fv = float(NE) * counts.astype(f32) / (float(TOPK) * T)
    aux_loss = jnp.sum(P * Fv)
    return (out.reshape(B, N, C), aux_loss, counts)


# speedup vs baseline: 1.3000x; 1.3000x over previous
import jax
import jax.numpy as jnp
from jax import lax
from jax.experimental import pallas as pl
from jax.experimental.pallas import tpu as pltpu
from jax.experimental.pallas import tpu_sc as plsc

D = 768
H = 12
HD = 64
HID = 576
NE = 23
TOPK = 3
NEP = 128
T = 2048
RB = 256
QB = 512
GB = 256
MAXBLKS = 47
MAXROWS = MAXBLKS * GB
NA = T * TOPK
NEG = -1e30

NC, NS = 2, 16
NW = NC * NS
PROWS = NA // NW
PCH = 96
PNCH = PROWS // PCH


def _ln(x, g, b, eps=1e-5):
    mu = jnp.mean(x, -1, keepdims=True)
    xc = x - mu
    var = jnp.mean(xc * xc, -1, keepdims=True)
    return xc * lax.rsqrt(var + eps) * g + b


def _gelu(x):
    return 0.5 * x * (1.0 + lax.erf(x * (2.0 ** -0.5)))


def _ln_qkv_kernel(x_ref, g_ref, b_ref, w_ref, o_ref):
    xa = _ln(x_ref[...], g_ref[...], b_ref[...]).astype(jnp.bfloat16)
    o_ref[...] = jnp.dot(xa, w_ref[...],
                         preferred_element_type=jnp.float32).astype(jnp.bfloat16)


def _attn_kernel(q_ref, kv_ref, o_ref):
    for h in range(H):
        q = q_ref[:, h * HD:(h + 1) * HD]
        k = kv_ref[:, D + h * HD:D + (h + 1) * HD]
        v = kv_ref[:, 2 * D + h * HD:2 * D + (h + 1) * HD]
        s = lax.dot_general(q, k, (((1,), (1,)), ((), ())),
                            preferred_element_type=jnp.float32) * (HD ** -0.5)
        m = jnp.max(s, -1, keepdims=True)
        e = jnp.exp(s - m)
        p = (e / jnp.sum(e, -1, keepdims=True)).astype(jnp.bfloat16)
        o_ref[:, h * HD:(h + 1) * HD] = jnp.dot(
            p, v, preferred_element_type=jnp.float32).astype(jnp.bfloat16)


def _proj_gate_kernel(ao_ref, x_ref, pw_ref, pb_ref, g2_ref, b2_ref,
                      gw_ref, gb_ref, x1_ref, xm_ref, gate_ref):
    x1 = (x_ref[...]
          + jnp.dot(ao_ref[...], pw_ref[...], preferred_element_type=jnp.float32)
          + pb_ref[...])
    x1_ref[...] = x1
    xm = _ln(x1, g2_ref[...], b2_ref[...])
    xm_ref[...] = xm
    logits = jnp.dot(xm, gw_ref[...], preferred_element_type=jnp.float32) + gb_ref[...]
    gate_ref[...] = jax.nn.sigmoid(logits)


def _router_kernel(gate_ref, bias_ref, idx_ref, w_ref, rank_ref, cnt_ref,
                   p_ref, run_ref):
    g = gate_ref[...]
    lane = lax.broadcasted_iota(jnp.int32, g.shape, 1)
    s = g + bias_ref[...]
    idxs, ws = [], []
    for _ in range(TOPK):
        m = jnp.max(s, -1, keepdims=True)
        hit = s == m
        idx = jnp.min(jnp.where(hit, lane, NEP), -1, keepdims=True)
        w = jnp.sum(jnp.where(lane == idx, g, 0.0), -1, keepdims=True)
        s = jnp.where(lane == idx, NEG, s)
        idxs.append(idx)
        ws.append(w)
    wsum = ws[0] + ws[1] + ws[2]
    idx_ref[...] = jnp.where(lane == 0, idxs[0],
                   jnp.where(lane == 1, idxs[1],
                   jnp.where(lane == 2, idxs[2], 0)))
    w_ref[...] = jnp.where(lane == 0, ws[0],
                 jnp.where(lane == 1, ws[1],
                 jnp.where(lane == 2, ws[2], 0.0))) / wsum

    @pl.when(pl.program_id(0) == 0)
    def _():
        run_ref[...] = jnp.zeros_like(run_ref)

    oh3 = ((lane == idxs[0]) | (lane == idxs[1]) | (lane == idxs[2]))
    oh3 = oh3.astype(jnp.bfloat16)
    r = lax.broadcasted_iota(jnp.int32, (RB, RB), 0)
    c = lax.broadcasted_iota(jnp.int32, (RB, RB), 1)
    ltri = (c < r).astype(jnp.bfloat16)
    cum = jnp.dot(ltri, oh3, preferred_element_type=jnp.float32)
    tot = cum + run_ref[...]
    ranks = [jnp.sum(jnp.where(lane == idxs[k], tot, 0.0), -1, keepdims=True)
             for k in range(TOPK)]
    rank_ref[...] = jnp.where(lane == 0, ranks[0],
                    jnp.where(lane == 1, ranks[1],
                    jnp.where(lane == 2, ranks[2], 0.0)))

    blockcnt = jnp.sum(oh3.astype(jnp.float32), 0, keepdims=True)
    run_ref[...] = run_ref[...] + blockcnt
    cnt_ref[0] = blockcnt
    gn = g / jnp.sum(g, -1, keepdims=True)
    p_ref[0] = jnp.sum(gn, 0, keepdims=True)


def _dest_kernel(idx_ref, rank_ref, po_ref, dga_ref):
    idxv = idx_ref[...]
    rankv = rank_ref[...]
    po = po_ref[...]
    lane = lax.broadcasted_iota(jnp.int32, idxv.shape, 1)
    dests = []
    for k in range(TOPK):
        ek = jnp.sum(jnp.where(lane == k, idxv, 0), -1, keepdims=True)
        rk = jnp.sum(jnp.where(lane == k, rankv, 0.0), -1, keepdims=True)
        pk = jnp.sum(jnp.where(lane == ek, po, 0.0), -1, keepdims=True)
        dests.append((pk + rk).astype(jnp.int32))
    dga_ref[...] = jnp.concatenate(dests, axis=1)


def _gmm_kernel(be_ref, xd_ref, w1_ref, b1_ref, w2_ref, b2_ref, o_ref):
    xb = xd_ref[...].astype(jnp.bfloat16)
    h = jnp.dot(xb, w1_ref[0], preferred_element_type=jnp.float32) + b1_ref[0]
    h = _gelu(h).astype(jnp.bfloat16)
    o_ref[...] = jnp.dot(h, w2_ref[0], preferred_element_type=jnp.float32) + b2_ref[0]


def _shared_kernel(xm_ref, x1_ref, w1_ref, b1_ref, w2_ref, b2_ref, o_ref):
    xb = xm_ref[...].astype(jnp.bfloat16)
    h = jnp.dot(xb, w1_ref[...], preferred_element_type=jnp.float32) + b1_ref[...]
    h = _gelu(h).astype(jnp.bfloat16)
    o_ref[...] = (x1_ref[...]
                  + jnp.dot(h, w2_ref[...], preferred_element_type=jnp.float32)
                  + b2_ref[...])


def _dispatch_body(xm_hbm, tok_hbm, dga_hbm, xd_hbm, idx_v, dst_v, rows_v, sem):
    c = lax.axis_index("c")
    s = lax.axis_index("s")
    base = (c * NS + s) * PROWS
    for j in range(PNCH):
        off = base + j * PCH
        pltpu.sync_copy(tok_hbm.at[pl.ds(off, PCH)], idx_v)
        pltpu.async_copy(xm_hbm.at[idx_v], rows_v, sem).wait()
        pltpu.sync_copy(dga_hbm.at[pl.ds(off, PCH)], dst_v)
        pltpu.async_copy(rows_v, xd_hbm.at[dst_v], sem).wait()


def _permute_body(eout_hbm, dga_hbm, ep_hbm, idx_v, rows_v, sem):
    c = lax.axis_index("c")
    s = lax.axis_index("s")
    base = (c * NS + s) * PROWS
    for j in range(PNCH):
        off = base + j * PCH
        pltpu.sync_copy(dga_hbm.at[pl.ds(off, PCH)], idx_v)
        pltpu.async_copy(eout_hbm.at[idx_v], rows_v, sem).wait()
        pltpu.sync_copy(rows_v, ep_hbm.at[pl.ds(off, PCH)])


def _comb_kernel(ep_ref, part_ref, w_ref, o_ref):
    e3 = ep_ref[...].reshape(RB, TOPK, D)
    w = w_ref[...]
    o_ref[...] = (part_ref[...]
                  + w[:, 0:1] * e3[:, 0]
                  + w[:, 1:2] * e3[:, 1]
                  + w[:, 2:3] * e3[:, 2])


def kernel(x, norm1_g, norm1_b, qkv_w, proj_w, proj_b, norm2_g, norm2_b,
           gate_w, gate_b, e1_w, e1_b, e2_w, e2_b, s1_w, s1_b, s2_w, s2_b,
           moe_bias):
    B, N, C = x.shape
    x2 = x.reshape(T, D)
    f32 = jnp.float32
    bf16 = jnp.bfloat16

    qkv = pl.pallas_call(
        _ln_qkv_kernel,
        grid=(T // RB,),
        in_specs=[
            pl.BlockSpec((RB, D), lambda i: (i, 0)),
            pl.BlockSpec((1, D), lambda i: (0, 0)),
            pl.BlockSpec((1, D), lambda i: (0, 0)),
            pl.BlockSpec((D, 3 * D), lambda i: (0, 0)),
        ],
        out_specs=pl.BlockSpec((RB, 3 * D), lambda i: (i, 0)),
        out_shape=jax.ShapeDtypeStruct((T, 3 * D), bf16),
    )(x2, norm1_g.reshape(1, D), norm1_b.reshape(1, D), qkv_w.T.astype(bf16))

    ao = pl.pallas_call(
        _attn_kernel,
        grid=(T // QB,),
        in_specs=[
            pl.BlockSpec((QB, 3 * D), lambda i: (i, 0)),
            pl.BlockSpec((T, 3 * D), lambda i: (0, 0)),
        ],
        out_specs=pl.BlockSpec((QB, D), lambda i: (i, 0)),
        out_shape=jax.ShapeDtypeStruct((T, D), bf16),
    )(qkv, qkv)

    gw = jnp.zeros((D, NEP), f32).at[:, :NE].set(gate_w.T)
    gb = jnp.full((1, NEP), NEG, f32).at[0, :NE].set(gate_b)
    x1, xm, gate = pl.pallas_call(
        _proj_gate_kernel,
        grid=(T // RB,),
        in_specs=[
            pl.BlockSpec((RB, D), lambda i: (i, 0)),
            pl.BlockSpec((RB, D), lambda i: (i, 0)),
            pl.BlockSpec((D, D), lambda i: (0, 0)),
            pl.BlockSpec((1, D), lambda i: (0, 0)),
            pl.BlockSpec((1, D), lambda i: (0, 0)),
            pl.BlockSpec((1, D), lambda i: (0, 0)),
            pl.BlockSpec((D, NEP), lambda i: (0, 0)),
            pl.BlockSpec((1, NEP), lambda i: (0, 0)),
        ],
        out_specs=[
            pl.BlockSpec((RB, D), lambda i: (i, 0)),
            pl.BlockSpec((RB, D), lambda i: (i, 0)),
            pl.BlockSpec((RB, NEP), lambda i: (i, 0)),
        ],
        out_shape=[
            jax.ShapeDtypeStruct((T, D), f32),
            jax.ShapeDtypeStruct((T, D), f32),
            jax.ShapeDtypeStruct((T, NEP), f32),
        ],
    )(ao, x2, proj_w.T.astype(bf16), proj_b.reshape(1, D), norm2_g.reshape(1, D),
      norm2_b.reshape(1, D), gw, gb)

    mb = jnp.full((1, NEP), NEG, f32).at[0, :NE].set(moe_bias)
    idx_o, w_o, rank_o, cnt_p, p_p = pl.pallas_call(
        _router_kernel,
        grid=(T // RB,),
        in_specs=[
            pl.BlockSpec((RB, NEP), lambda i: (i, 0)),
            pl.BlockSpec((1, NEP), lambda i: (0, 0)),
        ],
        out_specs=[
            pl.BlockSpec((RB, NEP), lambda i: (i, 0)),
            pl.BlockSpec((RB, NEP), lambda i: (i, 0)),
            pl.BlockSpec((RB, NEP), lambda i: (i, 0)),
            pl.BlockSpec((1, 1, NEP), lambda i: (i, 0, 0)),
            pl.BlockSpec((1, 1, NEP), lambda i: (i, 0, 0)),
        ],
        out_shape=[
            jax.ShapeDtypeStruct((T, NEP), jnp.int32),
            jax.ShapeDtypeStruct((T, NEP), f32),
            jax.ShapeDtypeStruct((T, NEP), f32),
            jax.ShapeDtypeStruct((T // RB, 1, NEP), f32),
            jax.ShapeDtypeStruct((T // RB, 1, NEP), f32),
        ],
        scratch_shapes=[pltpu.VMEM((1, NEP), f32)],
    )(gate, mb)

    counts = cnt_p[:, 0, :NE].sum(0).astype(jnp.int32)

    padc = ((counts + GB - 1) // GB) * GB
    incl = jnp.cumsum(padc)
    pad_off = jnp.concatenate([jnp.zeros((1,), jnp.int32), incl[:-1]])
    po128 = jnp.zeros((1, NEP), f32).at[0, :NE].set(pad_off.astype(f32))
    blk_start = jnp.arange(MAXBLKS, dtype=jnp.int32) * GB
    block_expert = jnp.minimum(
        jnp.searchsorted(incl, blk_start, side='right').astype(jnp.int32), NE - 1)

    dga = pl.pallas_call(
        _dest_kernel,
        grid=(T // RB,),
        in_specs=[
            pl.BlockSpec((RB, NEP), lambda i: (i, 0)),
            pl.BlockSpec((RB, NEP), lambda i: (i, 0)),
            pl.BlockSpec((1, NEP), lambda i: (0, 0)),
        ],
        out_specs=pl.BlockSpec((RB, TOPK), lambda i: (i, 0)),
        out_shape=jax.ShapeDtypeStruct((T, TOPK), jnp.int32),
    )(idx_o, rank_o, po128)

    dga_flat = dga.reshape(NA)
    tok3 = jnp.repeat(jnp.arange(T, dtype=jnp.int32), TOPK)

    mesh = plsc.VectorSubcoreMesh(core_axis_name="c", subcore_axis_name="s")

    xd = pl.kernel(
        _dispatch_body,
        out_type=jax.ShapeDtypeStruct((MAXROWS, D), f32),
        mesh=mesh,
        scratch_types=[
            pltpu.VMEM((PCH,), jnp.int32),
            pltpu.VMEM((PCH,), jnp.int32),
            pltpu.VMEM((PCH, D), f32),
            pltpu.SemaphoreType.DMA,
        ],
    )(xm, tok3, dga_flat)

    eout = pl.pallas_call(
        _gmm_kernel,
        grid_spec=pltpu.PrefetchScalarGridSpec(
            num_scalar_prefetch=1,
            grid=(MAXBLKS,),
            in_specs=[
                pl.BlockSpec((GB, D), lambda i, be: (i, 0)),
                pl.BlockSpec((1, D, HID), lambda i, be: (be[i], 0, 0)),
                pl.BlockSpec((1, 1, HID), lambda i, be: (be[i], 0, 0)),
                pl.BlockSpec((1, HID, D), lambda i, be: (be[i], 0, 0)),
                pl.BlockSpec((1, 1, D), lambda i, be: (be[i], 0, 0)),
            ],
            out_specs=pl.BlockSpec((GB, D), lambda i, be: (i, 0)),
        ),
        out_shape=jax.ShapeDtypeStruct((MAXROWS, D), f32),
    )(block_expert, xd, e1_w.transpose(0, 2, 1).astype(bf16),
      e1_b.reshape(NE, 1, HID), e2_w.transpose(0, 2, 1).astype(bf16),
      e2_b.reshape(NE, 1, D))

    part = pl.pallas_call(
        _shared_kernel,
        grid=(T // RB,),
        in_specs=[
            pl.BlockSpec((RB, D), lambda i: (i, 0)),
            pl.BlockSpec((RB, D), lambda i: (i, 0)),
            pl.BlockSpec((D, HID), lambda i: (0, 0)),
            pl.BlockSpec((1, HID), lambda i: (0, 0)),
            pl.BlockSpec((HID, D), lambda i: (0, 0)),
            pl.BlockSpec((1, D), lambda i: (0, 0)),
        ],
        out_specs=pl.BlockSpec((RB, D), lambda i: (i, 0)),
        out_shape=jax.ShapeDtypeStruct((T, D), f32),
    )(xm, x1, s1_w.T.astype(bf16), s1_b.reshape(1, HID),
      s2_w.T.astype(bf16), s2_b.reshape(1, D))

    eperm = pl.kernel(
        _permute_body,
        out_type=jax.ShapeDtypeStruct((NA, D), f32),
        mesh=mesh,
        scratch_types=[
            pltpu.VMEM((PCH,), jnp.int32),
            pltpu.VMEM((PCH, D), f32),
            pltpu.SemaphoreType.DMA,
        ],
    )(eout, dga_flat)

    out = pl.pallas_call(
        _comb_kernel,
        grid=(T // RB,),
        in_specs=[
            pl.BlockSpec((RB * TOPK, D), lambda i: (i, 0)),
            pl.BlockSpec((RB, D), lambda i: (i, 0)),
            pl.BlockSpec((RB, NEP), lambda i: (i, 0)),
        ],
        out_specs=pl.BlockSpec((RB, D), lambda i: (i, 0)),
        out_shape=jax.ShapeDtypeStruct((T, D), f32),
    )(eperm, part, w_o)

    P = p_p[:, 0, :NE].sum(0) / T
    Fv = float(NE) * counts.astype(f32) / (float(TOPK) * T)
    aux_loss = jnp.sum(P * Fv)
    return (out.reshape(B, N, C), aux_loss, counts)


# speedup vs baseline: 1.3079x; 1.0061x over previous
import jax
import jax.numpy as jnp
from jax import lax
from jax.experimental import pallas as pl
from jax.experimental.pallas import tpu as pltpu
from jax.experimental.pallas import tpu_sc as plsc

D = 768
H = 12
HD = 64
HID = 576
NE = 23
TOPK = 3
NEP = 128
T = 2048
RB = 256
QB = 512
GB = 384
MAXBLKS = 39
MAXROWS = MAXBLKS * GB
NA = T * TOPK
NEG = -1e30

NC, NS = 2, 16
NW = NC * NS
PROWS = NA // NW
PCH = 96
PNCH = PROWS // PCH


def _ln(x, g, b, eps=1e-5):
    mu = jnp.mean(x, -1, keepdims=True)
    xc = x - mu
    var = jnp.mean(xc * xc, -1, keepdims=True)
    return xc * lax.rsqrt(var + eps) * g + b


def _gelu(x):
    return 0.5 * x * (1.0 + lax.erf(x * (2.0 ** -0.5)))


def _ln_qkv_kernel(x_ref, g_ref, b_ref, w_ref, o_ref):
    xa = _ln(x_ref[...], g_ref[...], b_ref[...]).astype(jnp.bfloat16)
    o_ref[...] = jnp.dot(xa, w_ref[...],
                         preferred_element_type=jnp.float32).astype(jnp.bfloat16)


def _attn_kernel(q_ref, kv_ref, o_ref):
    for h in range(H):
        q = q_ref[:, h * HD:(h + 1) * HD]
        k = kv_ref[:, D + h * HD:D + (h + 1) * HD]
        v = kv_ref[:, 2 * D + h * HD:2 * D + (h + 1) * HD]
        s = lax.dot_general(q, k, (((1,), (1,)), ((), ())),
                            preferred_element_type=jnp.float32) * (HD ** -0.5)
        m = jnp.max(s, -1, keepdims=True)
        e = jnp.exp(s - m)
        p = (e / jnp.sum(e, -1, keepdims=True)).astype(jnp.bfloat16)
        o_ref[:, h * HD:(h + 1) * HD] = jnp.dot(
            p, v, preferred_element_type=jnp.float32).astype(jnp.bfloat16)


def _proj_gate_kernel(ao_ref, x_ref, pw_ref, pb_ref, g2_ref, b2_ref,
                      gw_ref, gb_ref, x1_ref, xm_ref, gate_ref):
    x1 = (x_ref[...]
          + jnp.dot(ao_ref[...], pw_ref[...], preferred_element_type=jnp.float32)
          + pb_ref[...])
    x1_ref[...] = x1
    xm = _ln(x1, g2_ref[...], b2_ref[...])
    xm_ref[...] = xm
    logits = jnp.dot(xm, gw_ref[...], preferred_element_type=jnp.float32) + gb_ref[...]
    gate_ref[...] = jax.nn.sigmoid(logits)


def _router_kernel(gate_ref, bias_ref, idx_ref, w_ref, rank_ref, cnt_ref,
                   p_ref, run_ref):
    g = gate_ref[...]
    lane = lax.broadcasted_iota(jnp.int32, g.shape, 1)
    s = g + bias_ref[...]
    idxs, ws = [], []
    for _ in range(TOPK):
        m = jnp.max(s, -1, keepdims=True)
        hit = s == m
        idx = jnp.min(jnp.where(hit, lane, NEP), -1, keepdims=True)
        w = jnp.sum(jnp.where(lane == idx, g, 0.0), -1, keepdims=True)
        s = jnp.where(lane == idx, NEG, s)
        idxs.append(idx)
        ws.append(w)
    wsum = ws[0] + ws[1] + ws[2]
    idx_ref[...] = jnp.where(lane == 0, idxs[0],
                   jnp.where(lane == 1, idxs[1],
                   jnp.where(lane == 2, idxs[2], 0)))
    w_ref[...] = jnp.where(lane == 0, ws[0],
                 jnp.where(lane == 1, ws[1],
                 jnp.where(lane == 2, ws[2], 0.0))) / wsum

    @pl.when(pl.program_id(0) == 0)
    def _():
        run_ref[...] = jnp.zeros_like(run_ref)

    oh3 = ((lane == idxs[0]) | (lane == idxs[1]) | (lane == idxs[2]))
    oh3 = oh3.astype(jnp.bfloat16)
    r = lax.broadcasted_iota(jnp.int32, (RB, RB), 0)
    c = lax.broadcasted_iota(jnp.int32, (RB, RB), 1)
    ltri = (c < r).astype(jnp.bfloat16)
    cum = jnp.dot(ltri, oh3, preferred_element_type=jnp.float32)
    tot = cum + run_ref[...]
    ranks = [jnp.sum(jnp.where(lane == idxs[k], tot, 0.0), -1, keepdims=True)
             for k in range(TOPK)]
    rank_ref[...] = jnp.where(lane == 0, ranks[0],
                    jnp.where(lane == 1, ranks[1],
                    jnp.where(lane == 2, ranks[2], 0.0)))

    blockcnt = jnp.sum(oh3.astype(jnp.float32), 0, keepdims=True)
    run_ref[...] = run_ref[...] + blockcnt
    cnt_ref[0] = blockcnt
    gn = g / jnp.sum(g, -1, keepdims=True)
    p_ref[0] = jnp.sum(gn, 0, keepdims=True)


def _dest_kernel(idx_ref, rank_ref, po_ref, dga_ref):
    idxv = idx_ref[...]
    rankv = rank_ref[...]
    po = po_ref[...]
    lane = lax.broadcasted_iota(jnp.int32, idxv.shape, 1)
    dests = []
    for k in range(TOPK):
        ek = jnp.sum(jnp.where(lane == k, idxv, 0), -1, keepdims=True)
        rk = jnp.sum(jnp.where(lane == k, rankv, 0.0), -1, keepdims=True)
        pk = jnp.sum(jnp.where(lane == ek, po, 0.0), -1, keepdims=True)
        dests.append((pk + rk).astype(jnp.int32))
    dga_ref[...] = jnp.concatenate(dests, axis=1)


def _gmm_kernel(be_ref, xd_ref, w1_ref, b1_ref, w2_ref, b2_ref, o_ref):
    xb = xd_ref[...].astype(jnp.bfloat16)
    h = jnp.dot(xb, w1_ref[0], preferred_element_type=jnp.float32) + b1_ref[0]
    h = _gelu(h).astype(jnp.bfloat16)
    o_ref[...] = jnp.dot(h, w2_ref[0], preferred_element_type=jnp.float32) + b2_ref[0]


def _shared_kernel(xm_ref, x1_ref, w1_ref, b1_ref, w2_ref, b2_ref, o_ref):
    xb = xm_ref[...].astype(jnp.bfloat16)
    h = jnp.dot(xb, w1_ref[...], preferred_element_type=jnp.float32) + b1_ref[...]
    h = _gelu(h).astype(jnp.bfloat16)
    o_ref[...] = (x1_ref[...]
                  + jnp.dot(h, w2_ref[...], preferred_element_type=jnp.float32)
                  + b2_ref[...])


def _dispatch_body(xm_hbm, tok_hbm, dga_hbm, xd_hbm, idx_v, dst_v, rows_v, sem):
    c = lax.axis_index("c")
    s = lax.axis_index("s")
    base = (c * NS + s) * PROWS
    for j in range(PNCH):
        off = base + j * PCH
        pltpu.sync_copy(tok_hbm.at[pl.ds(off, PCH)], idx_v)
        pltpu.async_copy(xm_hbm.at[idx_v], rows_v, sem).wait()
        pltpu.sync_copy(dga_hbm.at[pl.ds(off, PCH)], dst_v)
        pltpu.async_copy(rows_v, xd_hbm.at[dst_v], sem).wait()


def _permute_body(eout_hbm, dga_hbm, ep_hbm, idx_v, rows_v, sem):
    c = lax.axis_index("c")
    s = lax.axis_index("s")
    base = (c * NS + s) * PROWS
    for j in range(PNCH):
        off = base + j * PCH
        pltpu.sync_copy(dga_hbm.at[pl.ds(off, PCH)], idx_v)
        pltpu.async_copy(eout_hbm.at[idx_v], rows_v, sem).wait()
        pltpu.sync_copy(rows_v, ep_hbm.at[pl.ds(off, PCH)])


def _comb_kernel(ep_ref, part_ref, w_ref, o_ref):
    e3 = ep_ref[...].reshape(RB, TOPK, D)
    w = w_ref[...]
    o_ref[...] = (part_ref[...]
                  + w[:, 0:1] * e3[:, 0]
                  + w[:, 1:2] * e3[:, 1]
                  + w[:, 2:3] * e3[:, 2])


def kernel(x, norm1_g, norm1_b, qkv_w, proj_w, proj_b, norm2_g, norm2_b,
           gate_w, gate_b, e1_w, e1_b, e2_w, e2_b, s1_w, s1_b, s2_w, s2_b,
           moe_bias):
    B, N, C = x.shape
    x2 = x.reshape(T, D)
    f32 = jnp.float32
    bf16 = jnp.bfloat16

    qkv = pl.pallas_call(
        _ln_qkv_kernel,
        grid=(T // RB,),
        in_specs=[
            pl.BlockSpec((RB, D), lambda i: (i, 0)),
            pl.BlockSpec((1, D), lambda i: (0, 0)),
            pl.BlockSpec((1, D), lambda i: (0, 0)),
            pl.BlockSpec((D, 3 * D), lambda i: (0, 0)),
        ],
        out_specs=pl.BlockSpec((RB, 3 * D), lambda i: (i, 0)),
        out_shape=jax.ShapeDtypeStruct((T, 3 * D), bf16),
    )(x2, norm1_g.reshape(1, D), norm1_b.reshape(1, D), qkv_w.T.astype(bf16))

    ao = pl.pallas_call(
        _attn_kernel,
        grid=(T // QB,),
        in_specs=[
            pl.BlockSpec((QB, 3 * D), lambda i: (i, 0)),
            pl.BlockSpec((T, 3 * D), lambda i: (0, 0)),
        ],
        out_specs=pl.BlockSpec((QB, D), lambda i: (i, 0)),
        out_shape=jax.ShapeDtypeStruct((T, D), bf16),
    )(qkv, qkv)

    gw = jnp.zeros((D, NEP), f32).at[:, :NE].set(gate_w.T)
    gb = jnp.full((1, NEP), NEG, f32).at[0, :NE].set(gate_b)
    x1, xm, gate = pl.pallas_call(
        _proj_gate_kernel,
        grid=(T // RB,),
        in_specs=[
            pl.BlockSpec((RB, D), lambda i: (i, 0)),
            pl.BlockSpec((RB, D), lambda i: (i, 0)),
            pl.BlockSpec((D, D), lambda i: (0, 0)),
            pl.BlockSpec((1, D), lambda i: (0, 0)),
            pl.BlockSpec((1, D), lambda i: (0, 0)),
            pl.BlockSpec((1, D), lambda i: (0, 0)),
            pl.BlockSpec((D, NEP), lambda i: (0, 0)),
            pl.BlockSpec((1, NEP), lambda i: (0, 0)),
        ],
        out_specs=[
            pl.BlockSpec((RB, D), lambda i: (i, 0)),
            pl.BlockSpec((RB, D), lambda i: (i, 0)),
            pl.BlockSpec((RB, NEP), lambda i: (i, 0)),
        ],
        out_shape=[
            jax.ShapeDtypeStruct((T, D), f32),
            jax.ShapeDtypeStruct((T, D), f32),
            jax.ShapeDtypeStruct((T, NEP), f32),
        ],
    )(ao, x2, proj_w.T.astype(bf16), proj_b.reshape(1, D), norm2_g.reshape(1, D),
      norm2_b.reshape(1, D), gw, gb)

    mb = jnp.full((1, NEP), NEG, f32).at[0, :NE].set(moe_bias)
    idx_o, w_o, rank_o, cnt_p, p_p = pl.pallas_call(
        _router_kernel,
        grid=(T // RB,),
        in_specs=[
            pl.BlockSpec((RB, NEP), lambda i: (i, 0)),
            pl.BlockSpec((1, NEP), lambda i: (0, 0)),
        ],
        out_specs=[
            pl.BlockSpec((RB, NEP), lambda i: (i, 0)),
            pl.BlockSpec((RB, NEP), lambda i: (i, 0)),
            pl.BlockSpec((RB, NEP), lambda i: (i, 0)),
            pl.BlockSpec((1, 1, NEP), lambda i: (i, 0, 0)),
            pl.BlockSpec((1, 1, NEP), lambda i: (i, 0, 0)),
        ],
        out_shape=[
            jax.ShapeDtypeStruct((T, NEP), jnp.int32),
            jax.ShapeDtypeStruct((T, NEP), f32),
            jax.ShapeDtypeStruct((T, NEP), f32),
            jax.ShapeDtypeStruct((T // RB, 1, NEP), f32),
            jax.ShapeDtypeStruct((T // RB, 1, NEP), f32),
        ],
        scratch_shapes=[pltpu.VMEM((1, NEP), f32)],
    )(gate, mb)

    counts = cnt_p[:, 0, :NE].sum(0).astype(jnp.int32)

    padc = ((counts + GB - 1) // GB) * GB
    incl = jnp.cumsum(padc)
    pad_off = jnp.concatenate([jnp.zeros((1,), jnp.int32), incl[:-1]])
    po128 = jnp.zeros((1, NEP), f32).at[0, :NE].set(pad_off.astype(f32))
    blk_start = jnp.arange(MAXBLKS, dtype=jnp.int32) * GB
    block_expert = jnp.minimum(
        jnp.searchsorted(incl, blk_start, side='right').astype(jnp.int32), NE - 1)

    dga = pl.pallas_call(
        _dest_kernel,
        grid=(T // RB,),
        in_specs=[
            pl.BlockSpec((RB, NEP), lambda i: (i, 0)),
            pl.BlockSpec((RB, NEP), lambda i: (i, 0)),
            pl.BlockSpec((1, NEP), lambda i: (0, 0)),
        ],
        out_specs=pl.BlockSpec((RB, TOPK), lambda i: (i, 0)),
        out_shape=jax.ShapeDtypeStruct((T, TOPK), jnp.int32),
    )(idx_o, rank_o, po128)

    dga_flat = dga.reshape(NA)
    tok3 = jnp.repeat(jnp.arange(T, dtype=jnp.int32), TOPK)

    mesh = plsc.VectorSubcoreMesh(core_axis_name="c", subcore_axis_name="s")

    xd = pl.kernel(
        _dispatch_body,
        out_type=jax.ShapeDtypeStruct((MAXROWS, D), f32),
        mesh=mesh,
        scratch_types=[
            pltpu.VMEM((PCH,), jnp.int32),
            pltpu.VMEM((PCH,), jnp.int32),
            pltpu.VMEM((PCH, D), f32),
            pltpu.SemaphoreType.DMA,
        ],
    )(xm, tok3, dga_flat)

    eout = pl.pallas_call(
        _gmm_kernel,
        grid_spec=pltpu.PrefetchScalarGridSpec(
            num_scalar_prefetch=1,
            grid=(MAXBLKS,),
            in_specs=[
                pl.BlockSpec((GB, D), lambda i, be: (i, 0)),
                pl.BlockSpec((1, D, HID), lambda i, be: (be[i], 0, 0)),
                pl.BlockSpec((1, 1, HID), lambda i, be: (be[i], 0, 0)),
                pl.BlockSpec((1, HID, D), lambda i, be: (be[i], 0, 0)),
                pl.BlockSpec((1, 1, D), lambda i, be: (be[i], 0, 0)),
            ],
            out_specs=pl.BlockSpec((GB, D), lambda i, be: (i, 0)),
        ),
        out_shape=jax.ShapeDtypeStruct((MAXROWS, D), f32),
    )(block_expert, xd, e1_w.transpose(0, 2, 1).astype(bf16),
      e1_b.reshape(NE, 1, HID), e2_w.transpose(0, 2, 1).astype(bf16),
      e2_b.reshape(NE, 1, D))

    part = pl.pallas_call(
        _shared_kernel,
        grid=(T // RB,),
        in_specs=[
            pl.BlockSpec((RB, D), lambda i: (i, 0)),
            pl.BlockSpec((RB, D), lambda i: (i, 0)),
            pl.BlockSpec((D, HID), lambda i: (0, 0)),
            pl.BlockSpec((1, HID), lambda i: (0, 0)),
            pl.BlockSpec((HID, D), lambda i: (0, 0)),
            pl.BlockSpec((1, D), lambda i: (0, 0)),
        ],
        out_specs=pl.BlockSpec((RB, D), lambda i: (i, 0)),
        out_shape=jax.ShapeDtypeStruct((T, D), f32),
    )(xm, x1, s1_w.T.astype(bf16), s1_b.reshape(1, HID),
      s2_w.T.astype(bf16), s2_b.reshape(1, D))

    eperm = pl.kernel(
        _permute_body,
        out_type=jax.ShapeDtypeStruct((NA, D), f32),
        mesh=mesh,
        scratch_types=[
            pltpu.VMEM((PCH,), jnp.int32),
            pltpu.VMEM((PCH, D), f32),
            pltpu.SemaphoreType.DMA,
        ],
    )(eout, dga_flat)

    out = pl.pallas_call(
        _comb_kernel,
        grid=(T // RB,),
        in_specs=[
            pl.BlockSpec((RB * TOPK, D), lambda i: (i, 0)),
            pl.BlockSpec((RB, D), lambda i: (i, 0)),
            pl.BlockSpec((RB, NEP), lambda i: (i, 0)),
        ],
        out_specs=pl.BlockSpec((RB, D), lambda i: (i, 0)),
        out_shape=jax.ShapeDtypeStruct((T, D), f32),
    )(eperm, part, w_o)

    P = p_p[:, 0, :NE].sum(0) / T
    Fv = float(NE) * counts.astype(f32) / (float(TOPK) * T)
    aux_loss = jnp.sum(P * Fv)
    return (out.reshape(B, N, C), aux_loss, counts)
